# baseline (speedup 1.0000x reference)
import functools

import jax
import jax.numpy as jnp
from jax import lax
from jax.experimental import pallas as pl
from jax.experimental.pallas import tpu as pltpu
from jax.experimental.pallas import tpu_sc as plsc

NC = 2
NS = 16
NW = NC * NS
LANES = 16
CHUNK = 128


def _pad_kernel_make(e, e_pad, n, n_acc, dtype):
    def body(ei_ref, fill_ref, src_ref, dst_ref):
        fill = fill_ref[:]
        src_ref[pl.ds(0, e)] = ei_ref[0]
        src_ref[pl.ds(e, e_pad - e)] = fill
        dst_ref[pl.ds(0, e)] = ei_ref[1]
        dst_ref[pl.ds(e, e_pad - e)] = fill

    return pl.pallas_call(
        body,
        out_shape=(jax.ShapeDtypeStruct((e_pad,), dtype),
                   jax.ShapeDtypeStruct((e_pad,), dtype)),
    )


def _deg_kernel_make(n_acc, ept):
    mesh = plsc.VectorSubcoreMesh(
        core_axis_name="c", subcore_axis_name="s",
        num_cores=NC, num_subcores=NS)

    @functools.partial(
        pl.kernel,
        out_type=jax.ShapeDtypeStruct((NW, n_acc), jnp.float32),
        mesh=mesh,
        scratch_types=[
            pltpu.VMEM((n_acc,), jnp.float32),
            pltpu.VMEM((ept,), jnp.int32),
        ],
        compiler_params=pltpu.CompilerParams(needs_layout_passes=False),
    )
    def deg_kernel(dst_hbm, zeros_hbm, out_hbm, hist_v, idx_v):
        cid = lax.axis_index("c")
        sid = lax.axis_index("s")
        wid = sid * NC + cid

        pltpu.sync_copy(zeros_hbm, hist_v)
        pltpu.sync_copy(dst_hbm.at[wid], idx_v)

        ones = jnp.ones((LANES,), jnp.float32)

        def body(i, carry):
            idx = idx_v[pl.ds(i * LANES, LANES)]
            plsc.addupdate_scatter(hist_v, [idx], ones)
            return carry

        lax.fori_loop(0, ept // LANES, body, 0)
        pltpu.sync_copy(hist_v, out_hbm.at[wid])

    return deg_kernel


def _agg_kernel_make(n_acc, cpt, h):
    ept = cpt * CHUNK
    rpt = n_acc // NS
    mesh = plsc.VectorSubcoreMesh(
        core_axis_name="c", subcore_axis_name="s",
        num_cores=NC, num_subcores=NS)

    @functools.partial(
        pl.kernel,
        out_type=jax.ShapeDtypeStruct((NC, n_acc, h), jnp.float32),
        mesh=mesh,
        scratch_types=[
            pltpu.VMEM((CHUNK,), jnp.int32),
            pltpu.VMEM((CHUNK,), jnp.int32),
            pltpu.VMEM((CHUNK,), jnp.int32),
            pltpu.VMEM((CHUNK,), jnp.int32),
            pltpu.VMEM((CHUNK, h), jnp.float32),
            pltpu.VMEM((CHUNK, h), jnp.float32),
            pltpu.VMEM_SHARED((n_acc, h), jnp.float32),
            pltpu.SemaphoreType.DMA,
            pltpu.SemaphoreType.DMA,
        ],
        compiler_params=pltpu.CompilerParams(needs_layout_passes=False),
    )
    def agg_kernel(y_hbm, src_hbm, dst_hbm, zeros_hbm, out_hbm,
                   sidx0, sidx1, didx0, didx1, rows0, rows1,
                   agg_s, sem_i, sem_g):
        cid = lax.axis_index("c")
        sid = lax.axis_index("s")
        wid = sid * NC + cid
        sidx = (sidx0, sidx1)
        didx = (didx0, didx1)
        rows = (rows0, rows1)

        def idx_copies(c, b):
            base = wid * ept + c * CHUNK
            return (
                pltpu.make_async_copy(
                    src_hbm.at[pl.ds(base, CHUNK)], sidx[b], sem_i),
                pltpu.make_async_copy(
                    dst_hbm.at[pl.ds(base, CHUNK)], didx[b], sem_i),
            )

        def gather(b):
            return pltpu.make_async_copy(
                y_hbm.at[sidx[b]], rows[b], sem_g)

        pltpu.sync_copy(zeros_hbm, rows0)
        for k in range(rpt // CHUNK):
            pltpu.sync_copy(rows0,
                            agg_s.at[pl.ds(sid * rpt + k * CHUNK, CHUNK)])
        plsc.subcore_barrier()

        for cp in idx_copies(0, 0):
            cp.start()

        def pair_body(g, carry):
            for b in range(2):
                c = 2 * g + b
                for cp in idx_copies(c, b):
                    cp.wait()
                gather(b).start()

                @pl.when(c > 0)
                def _():
                    gather(1 - b).wait()
                    pltpu.sync_copy(rows[1 - b],
                                    agg_s.at[didx[1 - b]], add=True)

                @pl.when(c + 1 < cpt)
                def _():
                    for cp in idx_copies(c + 1, 1 - b):
                        cp.start()
            return carry

        lax.fori_loop(0, cpt // 2, pair_body, 0)
        gather(1).wait()
        pltpu.sync_copy(rows1, agg_s.at[didx1], add=True)
        plsc.subcore_barrier()

        for k in range(rpt // CHUNK):
            r0 = sid * rpt + k * CHUNK
            pltpu.sync_copy(agg_s.at[pl.ds(r0, CHUNK)], rows0)
            pltpu.sync_copy(rows0, out_hbm.at[cid, pl.ds(r0, CHUNK)])

    return agg_kernel


def _scale_kernel_make(n, n_acc, d, h):
    def body(x_ref, w_ref, p_ref, y_ref):
        xw = jnp.dot(x_ref[:], w_ref[:],
                     preferred_element_type=jnp.float32,
                     precision=lax.Precision.HIGHEST)
        ones = jnp.ones((NW, 1), jnp.float32)
        deg = lax.dot_general(p_ref[:], ones, (((0,), (0,)), ((), ())),
                              precision=lax.Precision.HIGHEST) + 1.0
        dis = lax.rsqrt(deg)
        y_ref[pl.ds(0, n)] = xw * dis[:n]
        y_ref[pl.ds(n, n_acc - n)] = jnp.zeros((n_acc - n, h), jnp.float32)

    return pl.pallas_call(
        body,
        out_shape=jax.ShapeDtypeStruct((n_acc, h), jnp.float32),
    )


def _merge_kernel_make(n, h):
    def body(pc_ref, y_ref, p_ref, b_ref, o_ref):
        ones = jnp.ones((NW, 1), jnp.float32)
        deg = lax.dot_general(p_ref[:], ones, (((0,), (0,)), ((), ())),
                              precision=lax.Precision.HIGHEST) + 1.0
        dis = lax.rsqrt(deg)
        s = pc_ref[0, :n] + pc_ref[1, :n] + y_ref[:n]
        o_ref[:] = jnp.maximum(s * dis[:n] + b_ref[:], 0.0)

    return pl.pallas_call(
        body,
        out_shape=jax.ShapeDtypeStruct((n, h), jnp.float32),
    )


def kernel(x, edge_index, W, b):
    n, d = x.shape
    h = W.shape[1]
    e = edge_index.shape[1]

    cpt = -(-e // (NW * CHUNK))
    cpt += cpt % 2
    e_pad = cpt * CHUNK * NW
    n_acc = -(-(n + 1) // (NS * CHUNK)) * (NS * CHUNK)

    fill = (n + jnp.arange(e_pad - e, dtype=edge_index.dtype)
            % jnp.asarray(n_acc - n, edge_index.dtype))
    src, dst = _pad_kernel_make(e, e_pad, n, n_acc,
                                edge_index.dtype)(edge_index, fill)
    dst2 = edge_index[1].reshape(NW, e // NW)
    zeros_stripe = jnp.zeros((CHUNK, h), jnp.float32)
    zeros_hist = jnp.zeros((n_acc,), jnp.float32)

    partials = _deg_kernel_make(n_acc, e // NW)(dst2, zeros_hist)
    y = _scale_kernel_make(n, n_acc, d, h)(x, W, partials)
    pc = _agg_kernel_make(n_acc, cpt, h)(y, src, dst, zeros_stripe)
    return _merge_kernel_make(n, h)(pc, y, partials, b[None, :])


# speedup vs baseline: 1.0745x; 1.0745x over previous
import functools

import jax
import jax.numpy as jnp
from jax import lax
from jax.experimental import pallas as pl
from jax.experimental.pallas import tpu as pltpu
from jax.experimental.pallas import tpu_sc as plsc

NC = 2
NS = 16
NW = NC * NS
LANES = 16
CHUNK = 128


def _pad_kernel_make(e, e_pad, n, n_acc, dtype):
    def body(ei_ref, fill_ref, src_ref, dst_ref):
        fill = fill_ref[:]
        src_ref[pl.ds(0, e)] = ei_ref[0]
        src_ref[pl.ds(e, e_pad - e)] = fill
        dst_ref[pl.ds(0, e)] = ei_ref[1]
        dst_ref[pl.ds(e, e_pad - e)] = fill

    return pl.pallas_call(
        body,
        out_shape=(jax.ShapeDtypeStruct((e_pad,), dtype),
                   jax.ShapeDtypeStruct((e_pad,), dtype)),
    )


def _deg_kernel_make(n_acc, cpt):
    mesh = plsc.VectorSubcoreMesh(
        core_axis_name="c", subcore_axis_name="s",
        num_cores=NC, num_subcores=NS)

    @functools.partial(
        pl.kernel,
        out_type=jax.ShapeDtypeStruct((NW, n_acc), jnp.float32),
        mesh=mesh,
        scratch_types=[
            pltpu.VMEM((n_acc,), jnp.float32),
            pltpu.VMEM((cpt, CHUNK), jnp.int32),
        ],
        compiler_params=pltpu.CompilerParams(needs_layout_passes=False),
    )
    def deg_kernel(dst_hbm, zeros_hbm, out_hbm, hist_v, idx_v):
        cid = lax.axis_index("c")
        sid = lax.axis_index("s")
        wid = sid * NC + cid

        pltpu.sync_copy(zeros_hbm, hist_v)
        pltpu.sync_copy(dst_hbm.at[wid], idx_v)

        ones = jnp.ones((LANES,), jnp.float32)

        def body(ci, carry):
            for j in range(CHUNK // LANES):
                idx = idx_v[ci, pl.ds(j * LANES, LANES)]
                plsc.addupdate_scatter(hist_v, [idx], ones)
            return carry

        lax.fori_loop(0, cpt, body, 0)
        pltpu.sync_copy(hist_v, out_hbm.at[wid])

    return deg_kernel


def _agg_kernel_make(n_acc, cpt, h):
    ept = cpt * CHUNK
    rpt = n_acc // NS
    mesh = plsc.VectorSubcoreMesh(
        core_axis_name="c", subcore_axis_name="s",
        num_cores=NC, num_subcores=NS)

    @functools.partial(
        pl.kernel,
        out_type=jax.ShapeDtypeStruct((NC, n_acc, h), jnp.float32),
        mesh=mesh,
        scratch_types=[
            pltpu.VMEM((CHUNK,), jnp.int32),
            pltpu.VMEM((CHUNK,), jnp.int32),
            pltpu.VMEM((CHUNK,), jnp.int32),
            pltpu.VMEM((CHUNK,), jnp.int32),
            pltpu.VMEM((CHUNK, h), jnp.float32),
            pltpu.VMEM((CHUNK, h), jnp.float32),
            pltpu.VMEM_SHARED((n_acc, h), jnp.float32),
            pltpu.SemaphoreType.DMA,
            pltpu.SemaphoreType.DMA,
        ],
        compiler_params=pltpu.CompilerParams(needs_layout_passes=False),
    )
    def agg_kernel(y_hbm, src_hbm, dst_hbm, zeros_hbm, out_hbm,
                   sidx0, sidx1, didx0, didx1, rows0, rows1,
                   agg_s, sem_i, sem_g):
        cid = lax.axis_index("c")
        sid = lax.axis_index("s")
        wid = sid * NC + cid
        sidx = (sidx0, sidx1)
        didx = (didx0, didx1)
        rows = (rows0, rows1)

        def idx_copies(c, b):
            base = wid * ept + c * CHUNK
            return (
                pltpu.make_async_copy(
                    src_hbm.at[pl.ds(base, CHUNK)], sidx[b], sem_i),
                pltpu.make_async_copy(
                    dst_hbm.at[pl.ds(base, CHUNK)], didx[b], sem_i),
            )

        def gather(b):
            return pltpu.make_async_copy(
                y_hbm.at[sidx[b]], rows[b], sem_g)

        pltpu.sync_copy(zeros_hbm, rows0)
        for k in range(rpt // CHUNK):
            pltpu.sync_copy(rows0,
                            agg_s.at[pl.ds(sid * rpt + k * CHUNK, CHUNK)])
        plsc.subcore_barrier()

        for cp in idx_copies(0, 0):
            cp.start()

        def pair_body(g, carry):
            for b in range(2):
                c = 2 * g + b
                for cp in idx_copies(c, b):
                    cp.wait()
                gather(b).start()

                @pl.when(c > 0)
                def _():
                    gather(1 - b).wait()
                    pltpu.sync_copy(rows[1 - b],
                                    agg_s.at[didx[1 - b]], add=True)

                @pl.when(c + 1 < cpt)
                def _():
                    for cp in idx_copies(c + 1, 1 - b):
                        cp.start()
            return carry

        lax.fori_loop(0, cpt // 2, pair_body, 0)
        gather(1).wait()
        pltpu.sync_copy(rows1, agg_s.at[didx1], add=True)
        plsc.subcore_barrier()

        for k in range(rpt // CHUNK):
            r0 = sid * rpt + k * CHUNK
            pltpu.sync_copy(agg_s.at[pl.ds(r0, CHUNK)], rows0)
            pltpu.sync_copy(rows0, out_hbm.at[cid, pl.ds(r0, CHUNK)])

    return agg_kernel


def _scale_kernel_make(n, n_acc, d, h):
    def body(x_ref, w_ref, p_ref, y_ref):
        xw = jnp.dot(x_ref[:], w_ref[:],
                     preferred_element_type=jnp.float32,
                     precision=lax.Precision.HIGHEST)
        ones = jnp.ones((NW, 1), jnp.float32)
        deg = lax.dot_general(p_ref[:], ones, (((0,), (0,)), ((), ())),
                              precision=lax.Precision.HIGHEST) + 1.0
        dis = lax.rsqrt(deg)
        y_ref[pl.ds(0, n)] = xw * dis[:n]
        y_ref[pl.ds(n, n_acc - n)] = jnp.zeros((n_acc - n, h), jnp.float32)

    return pl.pallas_call(
        body,
        out_shape=jax.ShapeDtypeStruct((n_acc, h), jnp.float32),
    )


def _merge_kernel_make(n, h):
    def body(pc_ref, y_ref, p_ref, b_ref, o_ref):
        ones = jnp.ones((NW, 1), jnp.float32)
        deg = lax.dot_general(p_ref[:], ones, (((0,), (0,)), ((), ())),
                              precision=lax.Precision.HIGHEST) + 1.0
        dis = lax.rsqrt(deg)
        s = pc_ref[0, :n] + pc_ref[1, :n] + y_ref[:n]
        o_ref[:] = jnp.maximum(s * dis[:n] + b_ref[:], 0.0)

    return pl.pallas_call(
        body,
        out_shape=jax.ShapeDtypeStruct((n, h), jnp.float32),
    )


def kernel(x, edge_index, W, b):
    n, d = x.shape
    h = W.shape[1]
    e = edge_index.shape[1]

    cpt = -(-e // (NW * CHUNK))
    cpt += cpt % 2
    e_pad = cpt * CHUNK * NW
    n_acc = -(-(n + 1) // (NS * CHUNK)) * (NS * CHUNK)

    fill = (n + jnp.arange(e_pad - e, dtype=edge_index.dtype)
            % jnp.asarray(n_acc - n, edge_index.dtype))
    src, dst = _pad_kernel_make(e, e_pad, n, n_acc,
                                edge_index.dtype)(edge_index, fill)
    dst3 = dst.reshape(NW, cpt, CHUNK)
    zeros_stripe = jnp.zeros((CHUNK, h), jnp.float32)
    zeros_hist = jnp.zeros((n_acc,), jnp.float32)

    partials = _deg_kernel_make(n_acc, cpt)(dst3, zeros_hist)
    y = _scale_kernel_make(n, n_acc, d, h)(x, W, partials)
    pc = _agg_kernel_make(n_acc, cpt, h)(y, src, dst, zeros_stripe)
    return _merge_kernel_make(n, h)(pc, y, partials, b[None, :])


# speedup vs baseline: 1.1851x; 1.1030x over previous
import functools

import jax
import jax.numpy as jnp
from jax import lax
from jax.experimental import pallas as pl
from jax.experimental.pallas import tpu as pltpu
from jax.experimental.pallas import tpu_sc as plsc

NC = 2
NS = 16
NW = NC * NS
LANES = 16
CHUNK = 128


def _pad_kernel_make(e, e_pad, n, n_acc, dtype):
    def body(ei_ref, fill_ref, src_ref, dst_ref):
        fill = fill_ref[:]
        src_ref[pl.ds(0, e)] = ei_ref[0]
        src_ref[pl.ds(e, e_pad - e)] = fill
        dst_ref[pl.ds(0, e)] = ei_ref[1]
        dst_ref[pl.ds(e, e_pad - e)] = fill

    return pl.pallas_call(
        body,
        out_shape=(jax.ShapeDtypeStruct((e_pad,), dtype),
                   jax.ShapeDtypeStruct((e_pad,), dtype)),
    )


def _deg_kernel_make(n_acc, cpt):
    mesh = plsc.VectorSubcoreMesh(
        core_axis_name="c", subcore_axis_name="s",
        num_cores=NC, num_subcores=NS)

    @functools.partial(
        pl.kernel,
        out_type=jax.ShapeDtypeStruct((NW, n_acc), jnp.float32),
        mesh=mesh,
        scratch_types=[
            pltpu.VMEM((n_acc,), jnp.float32),
            pltpu.VMEM((cpt, CHUNK), jnp.int32),
        ],
        compiler_params=pltpu.CompilerParams(needs_layout_passes=False),
    )
    def deg_kernel(dst_hbm, zeros_hbm, out_hbm, hist_v, idx_v):
        cid = lax.axis_index("c")
        sid = lax.axis_index("s")
        wid = sid * NC + cid

        pltpu.sync_copy(zeros_hbm, hist_v)
        pltpu.sync_copy(dst_hbm.at[wid], idx_v)

        ones = jnp.ones((LANES,), jnp.float32)

        def body(ci, carry):
            for j in range(CHUNK // LANES):
                idx = idx_v[ci, pl.ds(j * LANES, LANES)]
                plsc.addupdate_scatter(hist_v, [idx], ones)
            return carry

        lax.fori_loop(0, cpt, body, 0)
        pltpu.sync_copy(hist_v, out_hbm.at[wid])

    return deg_kernel


def _agg_kernel_make(n_acc, cpt, h):
    ept = cpt * CHUNK
    rpt = n_acc // NS
    mesh = plsc.VectorSubcoreMesh(
        core_axis_name="c", subcore_axis_name="s",
        num_cores=NC, num_subcores=NS)

    @functools.partial(
        pl.kernel,
        out_type=jax.ShapeDtypeStruct((NC, n_acc, h), jnp.float32),
        mesh=mesh,
        scratch_types=[
            pltpu.VMEM((CHUNK,), jnp.int32),
            pltpu.VMEM((CHUNK,), jnp.int32),
            pltpu.VMEM((CHUNK,), jnp.int32),
            pltpu.VMEM((CHUNK,), jnp.int32),
            pltpu.VMEM((CHUNK,), jnp.int32),
            pltpu.VMEM((CHUNK,), jnp.int32),
            pltpu.VMEM((CHUNK,), jnp.int32),
            pltpu.VMEM((CHUNK,), jnp.int32),
            pltpu.VMEM((CHUNK, h), jnp.float32),
            pltpu.VMEM((CHUNK, h), jnp.float32),
            pltpu.VMEM_SHARED((n_acc, h), jnp.float32),
            pltpu.SemaphoreType.DMA,
            pltpu.SemaphoreType.DMA,
            pltpu.SemaphoreType.DMA,
        ],
        compiler_params=pltpu.CompilerParams(needs_layout_passes=False),
    )
    def agg_kernel(y_hbm, src_hbm, dst_hbm, zeros_hbm, out_hbm,
                   sidx0, sidx1, sidx2, sidx3, didx0, didx1, didx2, didx3,
                   rows0, rows1, agg_s, sem_i, sem_g, sem_s):
        cid = lax.axis_index("c")
        sid = lax.axis_index("s")
        wid = sid * NC + cid
        sidx = (sidx0, sidx1, sidx2, sidx3)
        didx = (didx0, didx1, didx2, didx3)
        rows = (rows0, rows1)

        def idx_copies(c, q):
            base = wid * ept + c * CHUNK
            return (
                pltpu.make_async_copy(
                    src_hbm.at[pl.ds(base, CHUNK)], sidx[q], sem_i),
                pltpu.make_async_copy(
                    dst_hbm.at[pl.ds(base, CHUNK)], didx[q], sem_i),
            )

        def gath(p, q):
            return pltpu.make_async_copy(y_hbm.at[sidx[q]], rows[p], sem_g)

        def scat(p, q):
            return pltpu.make_async_copy(rows[p], agg_s.at[didx[q]], sem_s)

        pltpu.sync_copy(zeros_hbm, rows0)
        for k in range(rpt // CHUNK):
            pltpu.sync_copy(rows0,
                            agg_s.at[pl.ds(sid * rpt + k * CHUNK, CHUNK)])
        plsc.subcore_barrier()

        for cp in idx_copies(0, 0):
            cp.start()
        for cp in idx_copies(1, 1):
            cp.start()

        def quad_body(g, carry):
            for b in range(4):
                c = 4 * g + b
                p = b % 2
                for cp in idx_copies(c, b):
                    cp.wait()

                @pl.when(c > 1)
                def _():
                    scat(p, (b - 2) % 4).wait()

                gath(p, b).start()

                @pl.when(c > 0)
                def _():
                    gath(1 - p, (b - 1) % 4).wait()
                    scat(1 - p, (b - 1) % 4).start(add=True)

                @pl.when(c + 2 < cpt)
                def _():
                    for cp in idx_copies(c + 2, (b + 2) % 4):
                        cp.start()
            return carry

        lax.fori_loop(0, cpt // 4, quad_body, 0)
        gath(1, 3).wait()
        scat(1, 3).start(add=True)
        scat(0, 2).wait()
        scat(1, 3).wait()
        plsc.subcore_barrier()

        for k in range(rpt // CHUNK):
            r0 = sid * rpt + k * CHUNK
            pltpu.sync_copy(agg_s.at[pl.ds(r0, CHUNK)], rows0)
            pltpu.sync_copy(rows0, out_hbm.at[cid, pl.ds(r0, CHUNK)])

    return agg_kernel


def _scale_kernel_make(n, n_acc, d, h):
    def body(x_ref, w_ref, p_ref, y_ref):
        xw = jnp.dot(x_ref[:], w_ref[:],
                     preferred_element_type=jnp.float32,
                     precision=lax.Precision.HIGHEST)
        ones = jnp.ones((NW, 1), jnp.float32)
        deg = lax.dot_general(p_ref[:], ones, (((0,), (0,)), ((), ())),
                              precision=lax.Precision.HIGHEST) + 1.0
        dis = lax.rsqrt(deg)
        y_ref[pl.ds(0, n)] = xw * dis[:n]
        y_ref[pl.ds(n, n_acc - n)] = jnp.zeros((n_acc - n, h), jnp.float32)

    return pl.pallas_call(
        body,
        out_shape=jax.ShapeDtypeStruct((n_acc, h), jnp.float32),
    )


def _merge_kernel_make(n, h):
    def body(pc_ref, y_ref, p_ref, b_ref, o_ref):
        ones = jnp.ones((NW, 1), jnp.float32)
        deg = lax.dot_general(p_ref[:], ones, (((0,), (0,)), ((), ())),
                              precision=lax.Precision.HIGHEST) + 1.0
        dis = lax.rsqrt(deg)
        s = pc_ref[0, :n] + pc_ref[1, :n] + y_ref[:n]
        o_ref[:] = jnp.maximum(s * dis[:n] + b_ref[:], 0.0)

    return pl.pallas_call(
        body,
        out_shape=jax.ShapeDtypeStruct((n, h), jnp.float32),
    )


def kernel(x, edge_index, W, b):
    n, d = x.shape
    h = W.shape[1]
    e = edge_index.shape[1]

    cpt = -(-e // (NW * CHUNK))
    cpt += (-cpt) % 4
    e_pad = cpt * CHUNK * NW
    n_acc = -(-(n + 1) // (NS * CHUNK)) * (NS * CHUNK)

    fill = (n + jnp.arange(e_pad - e, dtype=edge_index.dtype)
            % jnp.asarray(n_acc - n, edge_index.dtype))
    src, dst = _pad_kernel_make(e, e_pad, n, n_acc,
                                edge_index.dtype)(edge_index, fill)
    dst3 = dst.reshape(NW, cpt, CHUNK)
    zeros_stripe = jnp.zeros((CHUNK, h), jnp.float32)
    zeros_hist = jnp.zeros((n_acc,), jnp.float32)

    partials = _deg_kernel_make(n_acc, cpt)(dst3, zeros_hist)
    y = _scale_kernel_make(n, n_acc, d, h)(x, W, partials)
    pc = _agg_kernel_make(n_acc, cpt, h)(y, src, dst, zeros_stripe)
    return _merge_kernel_make(n, h)(pc, y, partials, b[None, :])


# speedup vs baseline: 1.2316x; 1.0392x over previous
import functools

import jax
import jax.numpy as jnp
from jax import lax
from jax.experimental import pallas as pl
from jax.experimental.pallas import tpu as pltpu
from jax.experimental.pallas import tpu_sc as plsc

NC = 2
NS = 16
NW = NC * NS
LANES = 16
CHUNK = 128


def _pad_kernel_make(e, e_pad, n, n_acc, dtype):
    def body(ei_ref, fill_ref, src_ref, dst_ref):
        fill = fill_ref[:]
        src_ref[pl.ds(0, e)] = ei_ref[0]
        src_ref[pl.ds(e, e_pad - e)] = fill
        dst_ref[pl.ds(0, e)] = ei_ref[1]
        dst_ref[pl.ds(e, e_pad - e)] = fill

    return pl.pallas_call(
        body,
        out_shape=(jax.ShapeDtypeStruct((e_pad,), dtype),
                   jax.ShapeDtypeStruct((e_pad,), dtype)),
    )


def _deg_kernel_make(n_acc, cpt):
    mesh = plsc.VectorSubcoreMesh(
        core_axis_name="c", subcore_axis_name="s",
        num_cores=NC, num_subcores=NS)

    @functools.partial(
        pl.kernel,
        out_type=jax.ShapeDtypeStruct((NW, n_acc), jnp.float32),
        mesh=mesh,
        scratch_types=[
            pltpu.VMEM((n_acc,), jnp.float32),
            pltpu.VMEM((cpt, CHUNK), jnp.int32),
        ],
        compiler_params=pltpu.CompilerParams(needs_layout_passes=False),
    )
    def deg_kernel(dst_hbm, zeros_hbm, out_hbm, hist_v, idx_v):
        cid = lax.axis_index("c")
        sid = lax.axis_index("s")
        wid = sid * NC + cid

        pltpu.sync_copy(zeros_hbm, hist_v)
        pltpu.sync_copy(dst_hbm.at[wid], idx_v)

        ones = jnp.ones((LANES,), jnp.float32)

        def body(ci, carry):
            for j in range(CHUNK // LANES):
                idx = idx_v[ci, pl.ds(j * LANES, LANES)]
                plsc.addupdate_scatter(hist_v, [idx], ones)
            return carry

        lax.fori_loop(0, cpt, body, 0)
        pltpu.sync_copy(hist_v, out_hbm.at[wid])

    return deg_kernel


def _agg_kernel_make(n_acc, cpt, h):
    ept = cpt * CHUNK
    rpt = n_acc // NS
    mesh = plsc.VectorSubcoreMesh(
        core_axis_name="c", subcore_axis_name="s",
        num_cores=NC, num_subcores=NS)

    @functools.partial(
        pl.kernel,
        out_type=jax.ShapeDtypeStruct((NC, n_acc, h), jnp.float32),
        mesh=mesh,
        scratch_types=[
            pltpu.VMEM((CHUNK,), jnp.int32),
            pltpu.VMEM((CHUNK,), jnp.int32),
            pltpu.VMEM((CHUNK,), jnp.int32),
            pltpu.VMEM((CHUNK,), jnp.int32),
            pltpu.VMEM((CHUNK,), jnp.int32),
            pltpu.VMEM((CHUNK,), jnp.int32),
            pltpu.VMEM((CHUNK,), jnp.int32),
            pltpu.VMEM((CHUNK,), jnp.int32),
            pltpu.VMEM((CHUNK, h), jnp.float32),
            pltpu.VMEM((CHUNK, h), jnp.float32),
            pltpu.VMEM_SHARED((n_acc, h), jnp.float32),
            pltpu.SemaphoreType.DMA,
            pltpu.SemaphoreType.DMA,
            pltpu.SemaphoreType.DMA,
        ],
        compiler_params=pltpu.CompilerParams(needs_layout_passes=False),
    )
    def agg_kernel(y_hbm, src_hbm, dst_hbm, zeros_hbm, out_hbm,
                   sidx0, sidx1, sidx2, sidx3, didx0, didx1, didx2, didx3,
                   rows0, rows1, agg_s, sem_i, sem_g, sem_s):
        cid = lax.axis_index("c")
        sid = lax.axis_index("s")
        wid = sid * NC + cid
        sidx = (sidx0, sidx1, sidx2, sidx3)
        didx = (didx0, didx1, didx2, didx3)
        rows = (rows0, rows1)

        def idx_copies(c, q):
            base = wid * ept + c * CHUNK
            return (
                pltpu.make_async_copy(
                    src_hbm.at[pl.ds(base, CHUNK)], sidx[q], sem_i),
                pltpu.make_async_copy(
                    dst_hbm.at[pl.ds(base, CHUNK)], didx[q], sem_i),
            )

        def gath(p, q):
            return pltpu.make_async_copy(y_hbm.at[sidx[q]], rows[p], sem_g)

        def scat(p, q):
            return pltpu.make_async_copy(rows[p], agg_s.at[didx[q]], sem_s)

        for k in range(rpt // CHUNK):
            r0 = sid * rpt + k * CHUNK

            @pl.when(cid == 0)
            def _():
                pltpu.sync_copy(y_hbm.at[pl.ds(r0, CHUNK)], rows0)
                pltpu.sync_copy(rows0, agg_s.at[pl.ds(r0, CHUNK)])

        @pl.when(cid != 0)
        def _():
            pltpu.sync_copy(zeros_hbm, rows0)
            for k in range(rpt // CHUNK):
                pltpu.sync_copy(rows0,
                                agg_s.at[pl.ds(sid * rpt + k * CHUNK, CHUNK)])

        plsc.subcore_barrier()

        for cp in idx_copies(0, 0):
            cp.start()
        for cp in idx_copies(1, 1):
            cp.start()

        def quad_body(g, carry):
            for b in range(4):
                c = 4 * g + b
                p = b % 2
                for cp in idx_copies(c, b):
                    cp.wait()

                @pl.when(c > 1)
                def _():
                    scat(p, (b - 2) % 4).wait()

                gath(p, b).start()

                @pl.when(c > 0)
                def _():
                    gath(1 - p, (b - 1) % 4).wait()
                    scat(1 - p, (b - 1) % 4).start(add=True)

                @pl.when(c + 2 < cpt)
                def _():
                    for cp in idx_copies(c + 2, (b + 2) % 4):
                        cp.start()
            return carry

        lax.fori_loop(0, cpt // 4, quad_body, 0)
        gath(1, 3).wait()
        scat(1, 3).start(add=True)
        scat(0, 2).wait()
        scat(1, 3).wait()
        plsc.subcore_barrier()

        for k in range(rpt // CHUNK):
            r0 = sid * rpt + k * CHUNK
            pltpu.sync_copy(agg_s.at[pl.ds(r0, CHUNK)], rows0)
            pltpu.sync_copy(rows0, out_hbm.at[cid, pl.ds(r0, CHUNK)])

    return agg_kernel


def _scale_kernel_make(n, n_acc, d, h):
    def body(x_ref, w_ref, p_ref, y_ref):
        xw = jnp.dot(x_ref[:], w_ref[:],
                     preferred_element_type=jnp.float32)
        ones = jnp.ones((NW, 1), jnp.float32)
        deg = lax.dot_general(p_ref[:], ones,
                              (((0,), (0,)), ((), ()))) + 1.0
        dis = lax.rsqrt(deg)
        y_ref[pl.ds(0, n)] = xw * dis[:n]
        y_ref[pl.ds(n, n_acc - n)] = jnp.zeros((n_acc - n, h), jnp.float32)

    return pl.pallas_call(
        body,
        out_shape=jax.ShapeDtypeStruct((n_acc, h), jnp.float32),
    )


def _merge_kernel_make(n, h):
    def body(pc_ref, p_ref, b_ref, o_ref):
        ones = jnp.ones((NW, 1), jnp.float32)
        deg = lax.dot_general(p_ref[:], ones,
                              (((0,), (0,)), ((), ()))) + 1.0
        dis = lax.rsqrt(deg)
        s = pc_ref[0, :n] + pc_ref[1, :n]
        o_ref[:] = jnp.maximum(s * dis[:n] + b_ref[:], 0.0)

    return pl.pallas_call(
        body,
        out_shape=jax.ShapeDtypeStruct((n, h), jnp.float32),
    )


def kernel(x, edge_index, W, b):
    n, d = x.shape
    h = W.shape[1]
    e = edge_index.shape[1]

    cpt = -(-e // (NW * CHUNK))
    cpt += (-cpt) % 4
    e_pad = cpt * CHUNK * NW
    n_acc = -(-(n + 1) // (NS * CHUNK)) * (NS * CHUNK)

    fill = (n + jnp.arange(e_pad - e, dtype=edge_index.dtype)
            % jnp.asarray(n_acc - n, edge_index.dtype))
    src, dst = _pad_kernel_make(e, e_pad, n, n_acc,
                                edge_index.dtype)(edge_index, fill)
    dst3 = dst.reshape(NW, cpt, CHUNK)
    zeros_stripe = jnp.zeros((CHUNK, h), jnp.float32)
    zeros_hist = jnp.zeros((n_acc,), jnp.float32)

    partials = _deg_kernel_make(n_acc, cpt)(dst3, zeros_hist)
    y = _scale_kernel_make(n, n_acc, d, h)(x, W, partials)
    pc = _agg_kernel_make(n_acc, cpt, h)(y, src, dst, zeros_stripe)
    return _merge_kernel_make(n, h)(pc, partials, b[None, :])


# speedup vs baseline: 1.2561x; 1.0199x over previous
import functools

import jax
import jax.numpy as jnp
from jax import lax
from jax.experimental import pallas as pl
from jax.experimental.pallas import tpu as pltpu
from jax.experimental.pallas import tpu_sc as plsc

NC = 2
NS = 16
NW = NC * NS
LANES = 16
CHUNK = 128


def _pad_kernel_make(e, e_pad, n, n_acc, dtype):
    def body(ei_ref, fill_ref, src_ref, dst_ref):
        fill = fill_ref[:]
        src_ref[pl.ds(0, e)] = ei_ref[0]
        src_ref[pl.ds(e, e_pad - e)] = fill
        dst_ref[pl.ds(0, e)] = ei_ref[1]
        dst_ref[pl.ds(e, e_pad - e)] = fill

    return pl.pallas_call(
        body,
        out_shape=(jax.ShapeDtypeStruct((e_pad,), dtype),
                   jax.ShapeDtypeStruct((e_pad,), dtype)),
    )


def _deg_kernel_make(n_acc, cpt):
    mesh = plsc.VectorSubcoreMesh(
        core_axis_name="c", subcore_axis_name="s",
        num_cores=NC, num_subcores=NS)

    @functools.partial(
        pl.kernel,
        out_type=jax.ShapeDtypeStruct((NW, n_acc), jnp.float32),
        mesh=mesh,
        scratch_types=[
            pltpu.VMEM((n_acc,), jnp.float32),
            pltpu.VMEM((cpt, CHUNK), jnp.int32),
        ],
        compiler_params=pltpu.CompilerParams(needs_layout_passes=False),
    )
    def deg_kernel(dst_hbm, zeros_hbm, out_hbm, hist_v, idx_v):
        cid = lax.axis_index("c")
        sid = lax.axis_index("s")
        wid = sid * NC + cid

        pltpu.sync_copy(zeros_hbm, hist_v)
        pltpu.sync_copy(dst_hbm.at[wid], idx_v)

        ones = jnp.ones((LANES,), jnp.float32)

        def body(ci, carry):
            for j in range(CHUNK // LANES):
                idx = idx_v[ci, pl.ds(j * LANES, LANES)]
                plsc.addupdate_scatter(hist_v, [idx], ones)
            return carry

        lax.fori_loop(0, cpt, body, 0)
        pltpu.sync_copy(hist_v, out_hbm.at[wid])

    return deg_kernel


def _agg_kernel_make(n_acc, cpt, h):
    ept = cpt * CHUNK
    rpt = n_acc // NS
    mesh = plsc.VectorSubcoreMesh(
        core_axis_name="c", subcore_axis_name="s",
        num_cores=NC, num_subcores=NS)

    @functools.partial(
        pl.kernel,
        out_type=jax.ShapeDtypeStruct((NC, n_acc, h), jnp.float32),
        mesh=mesh,
        scratch_types=[
            pltpu.VMEM((CHUNK,), jnp.int32),
            pltpu.VMEM((CHUNK,), jnp.int32),
            pltpu.VMEM((CHUNK,), jnp.int32),
            pltpu.VMEM((CHUNK,), jnp.int32),
            pltpu.VMEM((CHUNK,), jnp.int32),
            pltpu.VMEM((CHUNK,), jnp.int32),
            pltpu.VMEM((CHUNK,), jnp.int32),
            pltpu.VMEM((CHUNK,), jnp.int32),
            pltpu.VMEM((CHUNK, h), jnp.float32),
            pltpu.VMEM((CHUNK, h), jnp.float32),
            pltpu.VMEM_SHARED((n_acc, h), jnp.float32),
            pltpu.SemaphoreType.DMA,
            pltpu.SemaphoreType.DMA,
            pltpu.SemaphoreType.DMA,
        ],
        compiler_params=pltpu.CompilerParams(needs_layout_passes=False),
    )
    def agg_kernel(y_hbm, src_hbm, dst_hbm, zeros_hbm, out_hbm,
                   sidx0, sidx1, sidx2, sidx3, didx0, didx1, didx2, didx3,
                   rows0, rows1, agg_s, sem_i, sem_g, sem_s):
        cid = lax.axis_index("c")
        sid = lax.axis_index("s")
        wid = sid * NC + cid
        sidx = (sidx0, sidx1, sidx2, sidx3)
        didx = (didx0, didx1, didx2, didx3)
        rows = (rows0, rows1)

        def idx_copies(c, q):
            base = wid * ept + c * CHUNK
            return (
                pltpu.make_async_copy(
                    src_hbm.at[pl.ds(base, CHUNK)], sidx[q], sem_i),
                pltpu.make_async_copy(
                    dst_hbm.at[pl.ds(base, CHUNK)], didx[q], sem_i),
            )

        def gath(p, q):
            return pltpu.make_async_copy(y_hbm.at[sidx[q]], rows[p], sem_g)

        def scat(p, q):
            return pltpu.make_async_copy(rows[p], agg_s.at[didx[q]], sem_s)

        @pl.when(cid == 0)
        def _():
            pltpu.sync_copy(y_hbm.at[pl.ds(sid * rpt, rpt)],
                            agg_s.at[pl.ds(sid * rpt, rpt)])

        @pl.when(cid != 0)
        def _():
            pltpu.sync_copy(zeros_hbm, rows0)
            for k in range(rpt // CHUNK):
                pltpu.sync_copy(rows0,
                                agg_s.at[pl.ds(sid * rpt + k * CHUNK, CHUNK)])

        plsc.subcore_barrier()

        for cp in idx_copies(0, 0):
            cp.start()
        for cp in idx_copies(1, 1):
            cp.start()

        def quad_body(g, carry):
            for b in range(4):
                c = 4 * g + b
                p = b % 2
                for cp in idx_copies(c, b):
                    cp.wait()

                @pl.when(c > 1)
                def _():
                    scat(p, (b - 2) % 4).wait()

                gath(p, b).start()

                @pl.when(c > 0)
                def _():
                    gath(1 - p, (b - 1) % 4).wait()
                    scat(1 - p, (b - 1) % 4).start(add=True)

                @pl.when(c + 2 < cpt)
                def _():
                    for cp in idx_copies(c + 2, (b + 2) % 4):
                        cp.start()
            return carry

        lax.fori_loop(0, cpt // 4, quad_body, 0)
        gath(1, 3).wait()
        scat(1, 3).start(add=True)
        scat(0, 2).wait()
        scat(1, 3).wait()
        plsc.subcore_barrier()

        pltpu.sync_copy(agg_s.at[pl.ds(sid * rpt, rpt)],
                        out_hbm.at[cid, pl.ds(sid * rpt, rpt)])

    return agg_kernel


def _scale_kernel_make(n, n_acc, d, h):
    def body(x_ref, w_ref, p_ref, y_ref):
        xw = jnp.dot(x_ref[:], w_ref[:],
                     preferred_element_type=jnp.float32)
        ones = jnp.ones((NW, 1), jnp.float32)
        deg = lax.dot_general(p_ref[:], ones,
                              (((0,), (0,)), ((), ()))) + 1.0
        dis = lax.rsqrt(deg)
        y_ref[pl.ds(0, n)] = xw * dis[:n]
        y_ref[pl.ds(n, n_acc - n)] = jnp.zeros((n_acc - n, h), jnp.float32)

    return pl.pallas_call(
        body,
        out_shape=jax.ShapeDtypeStruct((n_acc, h), jnp.float32),
    )


def _merge_kernel_make(n, h):
    def body(pc_ref, p_ref, b_ref, o_ref):
        ones = jnp.ones((NW, 1), jnp.float32)
        deg = lax.dot_general(p_ref[:], ones,
                              (((0,), (0,)), ((), ()))) + 1.0
        dis = lax.rsqrt(deg)
        s = pc_ref[0, :n] + pc_ref[1, :n]
        o_ref[:] = jnp.maximum(s * dis[:n] + b_ref[:], 0.0)

    return pl.pallas_call(
        body,
        out_shape=jax.ShapeDtypeStruct((n, h), jnp.float32),
    )


def kernel(x, edge_index, W, b):
    n, d = x.shape
    h = W.shape[1]
    e = edge_index.shape[1]

    cpt = -(-e // (NW * CHUNK))
    cpt += (-cpt) % 4
    e_pad = cpt * CHUNK * NW
    n_acc = -(-(n + 1) // (NS * CHUNK)) * (NS * CHUNK)

    fill = (n + jnp.arange(e_pad - e, dtype=edge_index.dtype)
            % jnp.asarray(n_acc - n, edge_index.dtype))
    src, dst = _pad_kernel_make(e, e_pad, n, n_acc,
                                edge_index.dtype)(edge_index, fill)
    dst3 = dst.reshape(NW, cpt, CHUNK)
    zeros_stripe = jnp.zeros((CHUNK, h), jnp.float32)
    zeros_hist = jnp.zeros((n_acc,), jnp.float32)

    partials = _deg_kernel_make(n_acc, cpt)(dst3, zeros_hist)
    y = _scale_kernel_make(n, n_acc, d, h)(x, W, partials)
    pc = _agg_kernel_make(n_acc, cpt, h)(y, src, dst, zeros_stripe)
    return _merge_kernel_make(n, h)(pc, partials, b[None, :])


# speedup vs baseline: 1.2632x; 1.0056x over previous
import functools

import jax
import jax.numpy as jnp
from jax import lax
from jax.experimental import pallas as pl
from jax.experimental.pallas import tpu as pltpu
from jax.experimental.pallas import tpu_sc as plsc

NC = 2
NS = 16
NW = NC * NS
LANES = 16
CHUNK = 128


def _pad_kernel_make(e, e_pad, n, n_acc, dtype):
    def body(ei_ref, fill_ref, src_ref, dst_ref):
        fill = fill_ref[:]
        src_ref[pl.ds(0, e)] = ei_ref[0]
        src_ref[pl.ds(e, e_pad - e)] = fill
        dst_ref[pl.ds(0, e)] = ei_ref[1]
        dst_ref[pl.ds(e, e_pad - e)] = fill

    return pl.pallas_call(
        body,
        out_shape=(jax.ShapeDtypeStruct((e_pad,), dtype),
                   jax.ShapeDtypeStruct((e_pad,), dtype)),
    )


def _deg_kernel_make(n_acc, cpt):
    mesh = plsc.VectorSubcoreMesh(
        core_axis_name="c", subcore_axis_name="s",
        num_cores=NC, num_subcores=NS)

    @functools.partial(
        pl.kernel,
        out_type=jax.ShapeDtypeStruct((NW, n_acc), jnp.float32),
        mesh=mesh,
        scratch_types=[
            pltpu.VMEM((n_acc,), jnp.float32),
            pltpu.VMEM((cpt, CHUNK), jnp.int32),
        ],
        compiler_params=pltpu.CompilerParams(needs_layout_passes=False),
    )
    def deg_kernel(dst_hbm, zeros_hbm, out_hbm, hist_v, idx_v):
        cid = lax.axis_index("c")
        sid = lax.axis_index("s")
        wid = sid * NC + cid

        pltpu.sync_copy(zeros_hbm, hist_v)
        pltpu.sync_copy(dst_hbm.at[wid], idx_v)

        ones = jnp.ones((LANES,), jnp.float32)

        def body(ci, carry):
            for j in range(CHUNK // LANES):
                idx = idx_v[ci, pl.ds(j * LANES, LANES)]
                plsc.addupdate_scatter(hist_v, [idx], ones)
            return carry

        lax.fori_loop(0, cpt, body, 0)
        pltpu.sync_copy(hist_v, out_hbm.at[wid])

    return deg_kernel


def _agg_kernel_make(n_acc, cpt, h):
    ept = cpt * CHUNK
    rpt = n_acc // NS
    mesh = plsc.VectorSubcoreMesh(
        core_axis_name="c", subcore_axis_name="s",
        num_cores=NC, num_subcores=NS)

    @functools.partial(
        pl.kernel,
        out_type=jax.ShapeDtypeStruct((NC, n_acc, h), jnp.float32),
        mesh=mesh,
        scratch_types=[
            pltpu.VMEM((CHUNK,), jnp.int32),
            pltpu.VMEM((CHUNK,), jnp.int32),
            pltpu.VMEM((CHUNK,), jnp.int32),
            pltpu.VMEM((CHUNK,), jnp.int32),
            pltpu.VMEM((CHUNK,), jnp.int32),
            pltpu.VMEM((CHUNK,), jnp.int32),
            pltpu.VMEM((CHUNK,), jnp.int32),
            pltpu.VMEM((CHUNK,), jnp.int32),
            pltpu.VMEM((CHUNK, h), jnp.float32),
            pltpu.VMEM((CHUNK, h), jnp.float32),
            pltpu.VMEM_SHARED((n_acc, h), jnp.float32),
            pltpu.SemaphoreType.DMA,
            pltpu.SemaphoreType.DMA,
            pltpu.SemaphoreType.DMA,
        ],
        compiler_params=pltpu.CompilerParams(needs_layout_passes=False),
    )
    def agg_kernel(y_hbm, src_hbm, dst_hbm, zeros_hbm, out_hbm,
                   sidx0, sidx1, sidx2, sidx3, didx0, didx1, didx2, didx3,
                   rows0, rows1, agg_s, sem_i, sem_g, sem_s):
        cid = lax.axis_index("c")
        sid = lax.axis_index("s")
        wid = sid * NC + cid
        sidx = (sidx0, sidx1, sidx2, sidx3)
        didx = (didx0, didx1, didx2, didx3)
        rows = (rows0, rows1)

        def idx_copies(c, q):
            base = wid * ept + c * CHUNK
            return (
                pltpu.make_async_copy(
                    src_hbm.at[pl.ds(base, CHUNK)], sidx[q], sem_i),
                pltpu.make_async_copy(
                    dst_hbm.at[pl.ds(base, CHUNK)], didx[q], sem_i),
            )

        def gath(p, q):
            return pltpu.make_async_copy(y_hbm.at[sidx[q]], rows[p], sem_g)

        def scat(p, q):
            return pltpu.make_async_copy(rows[p], agg_s.at[didx[q]], sem_s)

        @pl.when(cid == 0)
        def _():
            pltpu.sync_copy(y_hbm.at[pl.ds(sid * rpt, rpt)],
                            agg_s.at[pl.ds(sid * rpt, rpt)])

        @pl.when(cid != 0)
        def _():
            pltpu.sync_copy(zeros_hbm, rows0)
            for k in range(rpt // CHUNK):
                pltpu.sync_copy(rows0,
                                agg_s.at[pl.ds(sid * rpt + k * CHUNK, CHUNK)])

        plsc.subcore_barrier()

        for cp in idx_copies(0, 0):
            cp.start()
        for cp in idx_copies(1, 1):
            cp.start()

        def quad_body(g, carry):
            for b in range(4):
                c = 4 * g + b
                p = b % 2
                for cp in idx_copies(c, b):
                    cp.wait()

                @pl.when(c > 1)
                def _():
                    scat(p, (b - 2) % 4).wait()

                gath(p, b).start()

                @pl.when(c > 0)
                def _():
                    gath(1 - p, (b - 1) % 4).wait()
                    scat(1 - p, (b - 1) % 4).start(add=True)

                @pl.when(c + 2 < cpt)
                def _():
                    for cp in idx_copies(c + 2, (b + 2) % 4):
                        cp.start()
            return carry

        lax.fori_loop(0, cpt // 4, quad_body, 0)
        gath(1, 3).wait()
        scat(1, 3).start(add=True)
        scat(0, 2).wait()
        scat(1, 3).wait()
        plsc.subcore_barrier()

        pltpu.sync_copy(agg_s.at[pl.ds(sid * rpt, rpt)],
                        out_hbm.at[cid, pl.ds(sid * rpt, rpt)])

    return agg_kernel


def _scale_kernel_make(n, n_acc, d, h):
    def body(x_ref, w_ref, p_ref, y_ref, dis_ref):
        xw = jnp.dot(x_ref[:], w_ref[:],
                     preferred_element_type=jnp.float32)
        ones = jnp.ones((NW, 1), jnp.float32)
        deg = lax.dot_general(p_ref[:], ones,
                              (((0,), (0,)), ((), ()))) + 1.0
        dis = lax.rsqrt(deg)
        dis_ref[:] = dis
        y_ref[pl.ds(0, n)] = xw * dis[:n]
        y_ref[pl.ds(n, n_acc - n)] = jnp.zeros((n_acc - n, h), jnp.float32)

    return pl.pallas_call(
        body,
        out_shape=(jax.ShapeDtypeStruct((n_acc, h), jnp.float32),
                   jax.ShapeDtypeStruct((n_acc, 1), jnp.float32)),
    )


def _merge_kernel_make(n, n_acc, h, nb):
    bn = n // nb

    def body(pc_ref, dis_ref, b_ref, o_ref):
        o_ref[:] = jnp.maximum(
            (pc_ref[0] + pc_ref[1]) * dis_ref[:] + b_ref[:], 0.0)

    return pl.pallas_call(
        body,
        grid=(nb,),
        in_specs=[
            pl.BlockSpec((NC, bn, h), lambda i: (0, i, 0)),
            pl.BlockSpec((bn, 1), lambda i: (i, 0)),
            pl.BlockSpec((1, h), lambda i: (0, 0)),
        ],
        out_specs=pl.BlockSpec((bn, h), lambda i: (i, 0)),
        out_shape=jax.ShapeDtypeStruct((n, h), jnp.float32),
    )


def kernel(x, edge_index, W, b):
    n, d = x.shape
    h = W.shape[1]
    e = edge_index.shape[1]

    cpt = -(-e // (NW * CHUNK))
    cpt += (-cpt) % 4
    e_pad = cpt * CHUNK * NW
    n_acc = -(-(n + 1) // (NS * CHUNK)) * (NS * CHUNK)

    fill = (n + jnp.arange(e_pad - e, dtype=edge_index.dtype)
            % jnp.asarray(n_acc - n, edge_index.dtype))
    src, dst = _pad_kernel_make(e, e_pad, n, n_acc,
                                edge_index.dtype)(edge_index, fill)
    dst3 = dst.reshape(NW, cpt, CHUNK)
    zeros_stripe = jnp.zeros((CHUNK, h), jnp.float32)
    zeros_hist = jnp.zeros((n_acc,), jnp.float32)

    partials = _deg_kernel_make(n_acc, cpt)(dst3, zeros_hist)
    y, dis = _scale_kernel_make(n, n_acc, d, h)(x, W, partials)
    pc = _agg_kernel_make(n_acc, cpt, h)(y, src, dst, zeros_stripe)
    return _merge_kernel_make(n, n_acc, h, 5)(pc, dis, b[None, :])
